# baseline (speedup 1.0000x reference)
import functools

import jax
import jax.numpy as jnp
from jax import lax
from jax.experimental import pallas as pl
from jax.experimental.pallas import tpu as pltpu
from jax.experimental.pallas import tpu_sc as plsc

_BATCH = 16384
_FEAT = 32
_LANES = 16

_NC = 2
_NS = 16
_NW = _NC * _NS
_BPW = _BATCH // _NW
_CHUNK = 128
_NQ = _BPW // _CHUNK
_ROWS = _FEAT * _NQ
_SUB = _CHUNK // _LANES


@functools.cache
def _build():
    mesh = plsc.VectorSubcoreMesh(core_axis_name="c", subcore_axis_name="s")

    @functools.partial(
        pl.kernel,
        mesh=mesh,
        out_type=jax.ShapeDtypeStruct((_NW, _LANES), jnp.float32),
        scratch_types=[
            pltpu.VMEM((_ROWS, _CHUNK), jnp.int32),
            pltpu.VMEM((_ROWS, _CHUNK), jnp.float32),
            pltpu.VMEM((_ROWS, _CHUNK), jnp.float32),
            pltpu.VMEM((_LANES,), jnp.float32),
            pltpu.SemaphoreType.DMA,
            pltpu.SemaphoreType.DMA,
        ],
        compiler_params=pltpu.CompilerParams(needs_layout_passes=False),
    )
    def center_loss_partials(featf_hbm, pidx_hbm, ctr_hbm, out_hbm,
                             idx_v, ctr_v, feat_v, acc_v, gsem, fsem):
        wid = lax.axis_index("s") * _NC + lax.axis_index("c")

        pltpu.sync_copy(pidx_hbm.at[wid], idx_v)
        fcopy = pltpu.async_copy(featf_hbm.at[wid], feat_v, fsem)

        def fire(j, carry):
            pltpu.async_copy(ctr_hbm.at[idx_v.at[j]], ctr_v.at[j], gsem)
            return carry

        lax.fori_loop(0, _ROWS, fire, 0)

        def drain(j, carry):
            pltpu.make_async_copy(
                ctr_hbm.at[idx_v.at[j]], ctr_v.at[j], gsem).wait()
            return carry

        lax.fori_loop(0, _ROWS, drain, 0)
        fcopy.wait()

        zeros = jnp.zeros((_LANES,), jnp.float32)

        def step(j, acc):
            for h in range(_SUB):
                cv = ctr_v[j, pl.ds(h * _LANES, _LANES)]
                fv = feat_v[j, pl.ds(h * _LANES, _LANES)]
                d = fv - cv
                acc = acc + d * d
            return acc

        acc = lax.fori_loop(0, _ROWS, step, zeros)
        acc_v[...] = acc
        pltpu.sync_copy(acc_v, out_hbm.at[wid])

    return center_loss_partials


def kernel(features, labels, centers):
    labels = labels.astype(jnp.int32)
    lab3 = labels.reshape(_NW, 1, _NQ, _CHUNK)
    planes = (jnp.arange(_FEAT, dtype=jnp.int32)
              * centers.shape[0]).reshape(1, _FEAT, 1, 1)
    pidx = (lab3 + planes).reshape(_NW, _ROWS, _CHUNK)
    featf = (features.reshape(_NW, _NQ, _CHUNK, _FEAT)
             .transpose(0, 3, 1, 2).reshape(_NW, _ROWS, _CHUNK))
    ctr_flat = centers.T.reshape(-1)
    partials = _build()(featf, pidx, ctr_flat)
    return jnp.sum(partials) * (0.5 / _BATCH)


# speedup vs baseline: 18.3373x; 18.3373x over previous
import functools

import jax
import jax.numpy as jnp
from jax import lax
from jax.experimental import pallas as pl
from jax.experimental.pallas import tpu as pltpu
from jax.experimental.pallas import tpu_sc as plsc

_BATCH = 16384
_FEAT = 32
_LANES = 16

_NC = 2
_NS = 16
_NW = _NC * _NS
_BPW = _BATCH // _NW
_CHUNK = 128
_NQ = _BPW // _CHUNK
_ROWS = _FEAT * _NQ
_SUB = _CHUNK // _LANES

_NCLASS = 1000000
_LTILES = (_NCLASS + 127) // 128
_NSLAB = _FEAT // 8
_NTILES = _NSLAB * _LTILES
_FLAT = _NTILES * 1024

_TB = 32
_WPS = _NW // _NSLAB
_TPW = 977
_NB = -(-_TPW // _TB)
_RSMAX = _LTILES - 1 - _TB


@functools.cache
def _build_retile():
    mesh = plsc.VectorSubcoreMesh(core_axis_name="c", subcore_axis_name="s")

    @functools.partial(
        pl.kernel,
        mesh=mesh,
        out_type=jax.ShapeDtypeStruct((_NTILES, 8, 128), jnp.float32),
        scratch_types=[
            pltpu.VMEM((2, 8, _TB * 128), jnp.float32),
            pltpu.VMEM((_FEAT, 128), jnp.float32),
            pltpu.SemaphoreType.DMA,
            pltpu.SemaphoreType.DMA,
        ],
        compiler_params=pltpu.CompilerParams(needs_layout_passes=False),
    )
    def retile(ctrT_hbm, hi_hbm, out_hbm, buf_v, pbuf_v, rsem, wsem):
        wid = lax.axis_index("s") * _NC + lax.axis_index("c")
        slab = wid // _WPS
        ww = wid % _WPS
        srow = pl.multiple_of(slab * 8, 8)

        def rstart(b):
            return jnp.minimum(ww * _TPW + b * _TB, _RSMAX)

        def rsrc(b):
            col = pl.multiple_of(rstart(b) * 128, 128)
            return ctrT_hbm.at[0, pl.ds(srow, 8), pl.ds(col, _TB * 128)]

        pltpu.async_copy(rsrc(0), buf_v.at[0], rsem)

        def batch(b, carry):
            cur = b & 1
            pltpu.make_async_copy(rsrc(b), buf_v.at[cur], rsem).wait()

            @pl.when(b + 1 < _NB)
            def _():
                pltpu.async_copy(rsrc(b + 1), buf_v.at[1 - cur], rsem)

            t0 = slab * _LTILES + rstart(b)
            for i in range(_TB):
                pltpu.async_copy(
                    buf_v.at[cur, :, pl.ds(i * 128, 128)],
                    out_hbm.at[t0 + i], wsem)
            pltpu.make_async_copy(rsrc(b), buf_v.at[cur], wsem).wait()
            return carry

        lax.fori_loop(0, _NB, batch, 0)

        pltpu.sync_copy(hi_hbm, pbuf_v)
        pltpu.sync_copy(
            pbuf_v.at[pl.ds(srow, 8), :],
            out_hbm.at[slab * _LTILES + _LTILES - 1])

    return retile


@functools.cache
def _build_gather():
    mesh = plsc.VectorSubcoreMesh(core_axis_name="c", subcore_axis_name="s")

    @functools.partial(
        pl.kernel,
        mesh=mesh,
        out_type=jax.ShapeDtypeStruct((_NW, _LANES), jnp.float32),
        scratch_types=[
            pltpu.VMEM((_ROWS, _CHUNK), jnp.int32),
            pltpu.VMEM((_ROWS, _CHUNK), jnp.float32),
            pltpu.VMEM((_ROWS, _CHUNK), jnp.float32),
            pltpu.VMEM((_LANES,), jnp.float32),
            pltpu.SemaphoreType.DMA,
            pltpu.SemaphoreType.DMA,
        ],
        compiler_params=pltpu.CompilerParams(needs_layout_passes=False),
    )
    def center_loss_partials(featf_hbm, pidx_hbm, ctr_hbm, out_hbm,
                             idx_v, ctr_v, feat_v, acc_v, gsem, fsem):
        wid = lax.axis_index("s") * _NC + lax.axis_index("c")

        pltpu.sync_copy(pidx_hbm.at[wid], idx_v)
        fcopy = pltpu.async_copy(featf_hbm.at[wid], feat_v, fsem)

        def fire(j, carry):
            pltpu.async_copy(ctr_hbm.at[idx_v.at[j]], ctr_v.at[j], gsem)
            return carry

        lax.fori_loop(0, _ROWS, fire, 0)

        def drain(j, carry):
            pltpu.make_async_copy(
                ctr_hbm.at[idx_v.at[j]], ctr_v.at[j], gsem).wait()
            return carry

        lax.fori_loop(0, _ROWS, drain, 0)
        fcopy.wait()

        zeros = jnp.zeros((_LANES,), jnp.float32)

        def step(j, acc):
            for h in range(_SUB):
                cv = ctr_v[j, pl.ds(h * _LANES, _LANES)]
                fv = feat_v[j, pl.ds(h * _LANES, _LANES)]
                d = fv - cv
                acc = acc + d * d
            return acc

        acc = lax.fori_loop(0, _ROWS, step, zeros)
        acc_v[...] = acc
        pltpu.sync_copy(acc_v, out_hbm.at[wid])

    return center_loss_partials


def kernel(features, labels, centers):
    labels = labels.astype(jnp.int32)
    lab3 = labels.reshape(_NW, 1, _NQ, _CHUNK)
    c = jnp.arange(_FEAT, dtype=jnp.int32).reshape(1, _FEAT, 1, 1)
    base = (c // 8) * (_LTILES * 1024) + (c % 8) * 128
    pidx = (base + (lab3 >> 7) * 1024 + (lab3 & 127)).reshape(
        _NW, _ROWS, _CHUNK)
    featf = (features.reshape(_NW, _NQ, _CHUNK, _FEAT)
             .transpose(0, 3, 1, 2).reshape(_NW, _ROWS, _CHUNK))
    ctrT = centers.T[None]
    hi = jnp.pad(centers[(_LTILES - 1) * 128:].T, ((0, 0), (0, 64)))
    tiles = _build_retile()(ctrT, hi)
    ctr_flat = tiles.reshape(_FLAT)
    partials = _build_gather()(featf, pidx, ctr_flat)
    return jnp.sum(partials) * (0.5 / _BATCH)
